# baseline (device time: 90973 ns/iter reference)
import jax
import jax.numpy as jnp
from jax import lax
from jax.experimental import pallas as pl
from jax.experimental.pallas import tpu as pltpu

N_DEV = 16
B = 2
SQ_LOC = 128
D_MODEL = 512
HQ_LOC = 4
DH = 64
SKV = 128
D_CHUNK = HQ_LOC * DH

CW_HOPS = 8
CCW_HOPS = 7
CCW_BASE = 9


def kernel(x, Wq, K_ext, V_ext, Wo):
    W = jnp.concatenate([Wq, Wo.T], axis=1).astype(jnp.bfloat16)

    def body(x_ref, w_ref, k_raw, v_raw, out_ref,
             comm, k_ref, v_ref, cw_send, cw_recv, ccw_send, ccw_recv):
        my_pos = lax.axis_index("i")
        left = lax.rem(my_pos - 1 + N_DEV, N_DEV)
        right = lax.rem(my_pos + 1, N_DEV)

        barrier_sem = pltpu.get_barrier_semaphore()
        for nbr in (left, right):
            pl.semaphore_signal(
                barrier_sem, inc=1,
                device_id=(nbr,), device_id_type=pl.DeviceIdType.MESH,
            )
        pl.semaphore_wait(barrier_sem, 2)

        i_idx = lax.broadcasted_iota(jnp.int32, (SQ_LOC, SKV), 0)
        j_idx = lax.broadcasted_iota(jnp.int32, (SQ_LOC, SKV), 1)
        qb = my_pos * (SQ_LOC // 64) + i_idx // 64
        kb = j_idx // 64
        mask = (qb == kb) | (kb == 0) | (lax.rem(qb + kb, 3) == 0)
        x_bf = x_ref[...].astype(jnp.bfloat16)

        def cw_rdma(h):
            return pltpu.make_async_remote_copy(
                src_ref=comm.at[h - 1], dst_ref=comm.at[h],
                send_sem=cw_send.at[h - 1], recv_sem=cw_recv.at[h - 1],
                device_id=(right,), device_id_type=pl.DeviceIdType.MESH,
            )

        def ccw_rdma(h):
            return pltpu.make_async_remote_copy(
                src_ref=comm.at[CCW_BASE + h - 1], dst_ref=comm.at[CCW_BASE + h],
                send_sem=ccw_send.at[h - 1], recv_sem=ccw_recv.at[h - 1],
                device_id=(left,), device_id_type=pl.DeviceIdType.MESH,
            )

        def accumulate_chunk(c, w_c):
            wq_c = w_c[:, :D_CHUNK]
            wo_t = w_c[:, D_CHUNK:]
            kc = k_ref[pl.ds(c * HQ_LOC, HQ_LOC)]
            vc = v_ref[pl.ds(c * HQ_LOC, HQ_LOC)]
            for b in range(B):
                q_bf = jnp.dot(x_bf[b], wq_c,
                               preferred_element_type=jnp.float32
                               ).astype(jnp.bfloat16)
                ctx_heads = []
                for hh in range(HQ_LOC):
                    q_bh = q_bf[:, hh * DH:(hh + 1) * DH]
                    k_bh = kc[hh, b]
                    v_bh = vc[hh, b]
                    scores = lax.dot_general(
                        q_bh, k_bh, (((1,), (1,)), ((), ())),
                        preferred_element_type=jnp.float32) * 0.125
                    scores = jnp.where(mask, scores, -1e9)
                    m = jnp.max(scores, axis=-1, keepdims=True)
                    w = jnp.exp(scores - m)
                    w = (w / jnp.sum(w, axis=-1, keepdims=True)
                         ).astype(jnp.bfloat16)
                    ctx_heads.append(
                        jnp.dot(w, v_bh, preferred_element_type=jnp.float32))
                ctx_b = jnp.concatenate(ctx_heads, axis=1)
                out_ref[b] = out_ref[b] + lax.dot_general(
                    ctx_b.astype(jnp.bfloat16), wo_t, (((1,), (1,)), ((), ())),
                    preferred_element_type=jnp.float32)

        out_ref[...] = jnp.zeros((B, SQ_LOC, D_MODEL), jnp.float32)

        comm[0] = w_ref[...]
        comm[CCW_BASE] = w_ref[...]
        cw_rdma(1).start()
        ccw_rdma(1).start()
        k_ref[...] = jnp.transpose(k_raw[...], (2, 0, 1, 3)).astype(
            jnp.bfloat16)
        v_ref[...] = jnp.transpose(v_raw[...], (2, 0, 1, 3)).astype(
            jnp.bfloat16)
        accumulate_chunk(my_pos, comm[0])

        def hop(h, carry):
            cw_rdma(h).wait()

            @pl.when(h < CW_HOPS)
            def _():
                cw_rdma(h + 1).start()

            @pl.when(h <= CCW_HOPS)
            def _():
                ccw_rdma(h).wait()

                @pl.when(h < CCW_HOPS)
                def _():
                    ccw_rdma(h + 1).start()

            accumulate_chunk(lax.rem(my_pos - h + N_DEV, N_DEV), comm[h])

            @pl.when(h <= CCW_HOPS)
            def _():
                accumulate_chunk(lax.rem(my_pos + h, N_DEV),
                                 comm[CCW_BASE + h])

            return carry

        lax.fori_loop(1, CW_HOPS + 1, hop, 0)

    return pl.pallas_call(
        body,
        out_shape=jax.ShapeDtypeStruct((B, SQ_LOC, D_MODEL), jnp.float32),
        in_specs=[pl.BlockSpec(memory_space=pltpu.VMEM)] * 4,
        out_specs=pl.BlockSpec(memory_space=pltpu.VMEM),
        scratch_shapes=[
            pltpu.VMEM((CCW_BASE + CCW_HOPS + 1, D_MODEL, 2 * D_CHUNK),
                       jnp.bfloat16),
            pltpu.VMEM((N_DEV * HQ_LOC, B, SKV, DH), jnp.bfloat16),
            pltpu.VMEM((N_DEV * HQ_LOC, B, SKV, DH), jnp.bfloat16),
            pltpu.SemaphoreType.DMA((CW_HOPS,)),
            pltpu.SemaphoreType.DMA((CW_HOPS,)),
            pltpu.SemaphoreType.DMA((CCW_HOPS,)),
            pltpu.SemaphoreType.DMA((CCW_HOPS,)),
        ],
        compiler_params=pltpu.CompilerParams(collective_id=0),
    )(x, W, K_ext, V_ext)


# device time: 78892 ns/iter; 1.1531x vs baseline; 1.1531x over previous
import jax
import jax.numpy as jnp
from jax import lax
from jax.experimental import pallas as pl
from jax.experimental.pallas import tpu as pltpu

N_DEV = 16
B = 2
SQ_LOC = 128
D_MODEL = 512
HQ_LOC = 4
DH = 64
SKV = 128
D_CHUNK = HQ_LOC * DH

CW_HOPS = 8
CCW_HOPS = 7
CCW_BASE = 9


def kernel(x, Wq, K_ext, V_ext, Wo):
    K_t = jnp.transpose(K_ext, (2, 0, 1, 3)).astype(jnp.bfloat16)
    V_t = jnp.transpose(V_ext, (2, 0, 1, 3)).astype(jnp.bfloat16)
    W = jnp.concatenate([Wq, Wo.T], axis=1).astype(jnp.bfloat16)

    def body(x_ref, w_ref, k_ref, v_ref, out_ref,
             comm, cw_send, cw_recv, ccw_send, ccw_recv):
        my_pos = lax.axis_index("i")
        left = lax.rem(my_pos - 1 + N_DEV, N_DEV)
        right = lax.rem(my_pos + 1, N_DEV)

        barrier_sem = pltpu.get_barrier_semaphore()
        for nbr in (left, right):
            pl.semaphore_signal(
                barrier_sem, inc=1,
                device_id=(nbr,), device_id_type=pl.DeviceIdType.MESH,
            )
        pl.semaphore_wait(barrier_sem, 2)

        i_idx = lax.broadcasted_iota(jnp.int32, (SQ_LOC, SKV), 0)
        j_idx = lax.broadcasted_iota(jnp.int32, (SQ_LOC, SKV), 1)
        qb = my_pos * (SQ_LOC // 64) + i_idx // 64
        kb = j_idx // 64
        mask = (qb == kb) | (kb == 0) | (lax.rem(qb + kb, 3) == 0)
        x_bf = x_ref[...].astype(jnp.bfloat16)

        def cw_rdma(h):
            return pltpu.make_async_remote_copy(
                src_ref=comm.at[h - 1], dst_ref=comm.at[h],
                send_sem=cw_send.at[h - 1], recv_sem=cw_recv.at[h - 1],
                device_id=(right,), device_id_type=pl.DeviceIdType.MESH,
            )

        def ccw_rdma(h):
            return pltpu.make_async_remote_copy(
                src_ref=comm.at[CCW_BASE + h - 1], dst_ref=comm.at[CCW_BASE + h],
                send_sem=ccw_send.at[h - 1], recv_sem=ccw_recv.at[h - 1],
                device_id=(left,), device_id_type=pl.DeviceIdType.MESH,
            )

        def accumulate_chunk(c, w_c):
            wq_c = w_c[:, :D_CHUNK]
            wo_t = w_c[:, D_CHUNK:]
            kc = k_ref[pl.ds(c * HQ_LOC, HQ_LOC)]
            vc = v_ref[pl.ds(c * HQ_LOC, HQ_LOC)]
            for b in range(B):
                q_bf = jnp.dot(x_bf[b], wq_c,
                               preferred_element_type=jnp.float32
                               ).astype(jnp.bfloat16)
                ctx_heads = []
                for hh in range(HQ_LOC):
                    q_bh = q_bf[:, hh * DH:(hh + 1) * DH]
                    k_bh = kc[hh, b]
                    v_bh = vc[hh, b]
                    scores = lax.dot_general(
                        q_bh, k_bh, (((1,), (1,)), ((), ())),
                        preferred_element_type=jnp.float32) * 0.125
                    scores = jnp.where(mask, scores, -1e9)
                    m = jnp.max(scores, axis=-1, keepdims=True)
                    w = jnp.exp(scores - m)
                    w = (w / jnp.sum(w, axis=-1, keepdims=True)
                         ).astype(jnp.bfloat16)
                    ctx_heads.append(
                        jnp.dot(w, v_bh, preferred_element_type=jnp.float32))
                ctx_b = jnp.concatenate(ctx_heads, axis=1)
                out_ref[b] = out_ref[b] + lax.dot_general(
                    ctx_b.astype(jnp.bfloat16), wo_t, (((1,), (1,)), ((), ())),
                    preferred_element_type=jnp.float32)

        out_ref[...] = jnp.zeros((B, SQ_LOC, D_MODEL), jnp.float32)

        comm[0] = w_ref[...]
        comm[CCW_BASE] = w_ref[...]
        cw_rdma(1).start()
        ccw_rdma(1).start()
        accumulate_chunk(my_pos, comm[0])

        def hop(h, carry):
            cw_rdma(h).wait()

            @pl.when(h < CW_HOPS)
            def _():
                cw_rdma(h + 1).start()

            @pl.when(h <= CCW_HOPS)
            def _():
                ccw_rdma(h).wait()

                @pl.when(h < CCW_HOPS)
                def _():
                    ccw_rdma(h + 1).start()

            accumulate_chunk(lax.rem(my_pos - h + N_DEV, N_DEV), comm[h])

            @pl.when(h <= CCW_HOPS)
            def _():
                accumulate_chunk(lax.rem(my_pos + h, N_DEV),
                                 comm[CCW_BASE + h])

            return carry

        lax.fori_loop(1, CW_HOPS + 1, hop, 0)

    return pl.pallas_call(
        body,
        out_shape=jax.ShapeDtypeStruct((B, SQ_LOC, D_MODEL), jnp.float32),
        in_specs=[pl.BlockSpec(memory_space=pltpu.VMEM)] * 4,
        out_specs=pl.BlockSpec(memory_space=pltpu.VMEM),
        scratch_shapes=[
            pltpu.VMEM((CCW_BASE + CCW_HOPS + 1, D_MODEL, 2 * D_CHUNK),
                       jnp.bfloat16),
            pltpu.SemaphoreType.DMA((CW_HOPS,)),
            pltpu.SemaphoreType.DMA((CW_HOPS,)),
            pltpu.SemaphoreType.DMA((CCW_HOPS,)),
            pltpu.SemaphoreType.DMA((CCW_HOPS,)),
        ],
        compiler_params=pltpu.CompilerParams(collective_id=0),
    )(x, W, K_t, V_t)


# device time: 78450 ns/iter; 1.1596x vs baseline; 1.0056x over previous
import jax
import jax.numpy as jnp
from jax import lax
from jax.experimental import pallas as pl
from jax.experimental.pallas import tpu as pltpu

N_DEV = 16
B = 2
SQ_LOC = 128
D_MODEL = 512
HQ_LOC = 4
DH = 64
SKV = 128
D_CHUNK = HQ_LOC * DH

CW_HOPS = 8
CCW_HOPS = 7
CCW_BASE = 9


def kernel(x, Wq, K_ext, V_ext, Wo):
    K_t = jnp.transpose(K_ext, (2, 0, 1, 3)).astype(jnp.bfloat16)
    V_t = jnp.transpose(V_ext, (2, 0, 1, 3)).astype(jnp.bfloat16)
    W = jnp.concatenate([Wq, Wo.T], axis=1).astype(jnp.bfloat16)

    def body(x_ref, w_ref, k_ref, v_ref, out_ref,
             comm, cw_send, cw_recv, ccw_send, ccw_recv):
        my_pos = lax.axis_index("i")
        left = lax.rem(my_pos - 1 + N_DEV, N_DEV)
        right = lax.rem(my_pos + 1, N_DEV)

        barrier_sem = pltpu.get_barrier_semaphore()
        for nbr in (left, right):
            pl.semaphore_signal(
                barrier_sem, inc=1,
                device_id=(nbr,), device_id_type=pl.DeviceIdType.MESH,
            )
        pl.semaphore_wait(barrier_sem, 2)

        i_idx = lax.broadcasted_iota(jnp.int32, (SQ_LOC, SKV), 0)
        j_idx = lax.broadcasted_iota(jnp.int32, (SQ_LOC, SKV), 1)
        qb = my_pos * (SQ_LOC // 64) + i_idx // 64
        kb = j_idx // 64
        mask = (qb == kb) | (kb == 0) | (lax.rem(qb + kb, 3) == 0)
        x_bf = x_ref[...].astype(jnp.bfloat16)

        HALF = D_MODEL // 2

        def cw_rdma(h, s):
            return pltpu.make_async_remote_copy(
                src_ref=comm.at[h - 1, pl.ds(s * HALF, HALF)],
                dst_ref=comm.at[h, pl.ds(s * HALF, HALF)],
                send_sem=cw_send.at[h - 1, s], recv_sem=cw_recv.at[h - 1, s],
                device_id=(right,), device_id_type=pl.DeviceIdType.MESH,
            )

        def ccw_rdma(h, s):
            return pltpu.make_async_remote_copy(
                src_ref=comm.at[CCW_BASE + h - 1, pl.ds(s * HALF, HALF)],
                dst_ref=comm.at[CCW_BASE + h, pl.ds(s * HALF, HALF)],
                send_sem=ccw_send.at[h - 1, s], recv_sem=ccw_recv.at[h - 1, s],
                device_id=(left,), device_id_type=pl.DeviceIdType.MESH,
            )

        def accumulate_chunk(c, w_c):
            wq_c = w_c[:, :D_CHUNK]
            wo_t = w_c[:, D_CHUNK:]
            kc = k_ref[pl.ds(c * HQ_LOC, HQ_LOC)]
            vc = v_ref[pl.ds(c * HQ_LOC, HQ_LOC)]
            for b in range(B):
                q_bf = jnp.dot(x_bf[b], wq_c,
                               preferred_element_type=jnp.float32
                               ).astype(jnp.bfloat16)
                ctx_heads = []
                for hh in range(HQ_LOC):
                    q_bh = q_bf[:, hh * DH:(hh + 1) * DH]
                    k_bh = kc[hh, b]
                    v_bh = vc[hh, b]
                    scores = lax.dot_general(
                        q_bh, k_bh, (((1,), (1,)), ((), ())),
                        preferred_element_type=jnp.float32) * 0.125
                    scores = jnp.where(mask, scores, -1e9)
                    m = jnp.max(scores, axis=-1, keepdims=True)
                    w = jnp.exp(scores - m)
                    w = (w / jnp.sum(w, axis=-1, keepdims=True)
                         ).astype(jnp.bfloat16)
                    ctx_heads.append(
                        jnp.dot(w, v_bh, preferred_element_type=jnp.float32))
                ctx_b = jnp.concatenate(ctx_heads, axis=1)
                out_ref[b] = out_ref[b] + lax.dot_general(
                    ctx_b.astype(jnp.bfloat16), wo_t, (((1,), (1,)), ((), ())),
                    preferred_element_type=jnp.float32)

        out_ref[...] = jnp.zeros((B, SQ_LOC, D_MODEL), jnp.float32)

        comm[0] = w_ref[...]
        comm[CCW_BASE] = w_ref[...]
        for s in range(2):
            cw_rdma(1, s).start()
            ccw_rdma(1, s).start()
        accumulate_chunk(my_pos, comm[0])

        def hop(h, carry):
            for s in range(2):
                cw_rdma(h, s).wait()

                @pl.when(h < CW_HOPS)
                def _():
                    cw_rdma(h + 1, s).start()

            @pl.when(h <= CCW_HOPS)
            def _():
                for s in range(2):
                    ccw_rdma(h, s).wait()

                    @pl.when(h < CCW_HOPS)
                    def _():
                        ccw_rdma(h + 1, s).start()

            accumulate_chunk(lax.rem(my_pos - h + N_DEV, N_DEV), comm[h])

            @pl.when(h <= CCW_HOPS)
            def _():
                accumulate_chunk(lax.rem(my_pos + h, N_DEV),
                                 comm[CCW_BASE + h])

            return carry

        lax.fori_loop(1, CW_HOPS + 1, hop, 0)

    return pl.pallas_call(
        body,
        out_shape=jax.ShapeDtypeStruct((B, SQ_LOC, D_MODEL), jnp.float32),
        in_specs=[pl.BlockSpec(memory_space=pltpu.VMEM)] * 4,
        out_specs=pl.BlockSpec(memory_space=pltpu.VMEM),
        scratch_shapes=[
            pltpu.VMEM((CCW_BASE + CCW_HOPS + 1, D_MODEL, 2 * D_CHUNK),
                       jnp.bfloat16),
            pltpu.SemaphoreType.DMA((CW_HOPS, 2)),
            pltpu.SemaphoreType.DMA((CW_HOPS, 2)),
            pltpu.SemaphoreType.DMA((CCW_HOPS, 2)),
            pltpu.SemaphoreType.DMA((CCW_HOPS, 2)),
        ],
        compiler_params=pltpu.CompilerParams(collective_id=0),
    )(x, W, K_t, V_t)
